# baseline (speedup 1.0000x reference)
import jax
import jax.numpy as jnp
import numpy as np
from jax.experimental import pallas as pl

_B = 128
_T = 512
_D = 1024
_OUT = 96
_HID = 64
_K = 5
_INLEN = 512
_TB = 128
_LM = 4096


def _gelu(x):
    return 0.5 * x * (1.0 + jax.lax.erf(x * 0.7071067811865476))


def _stream_body(m_ref, w4_ref, wall_ref, cb_ref, pool_ref):
    h = m_ref[0]
    f32 = jnp.float32

    s = jnp.sum(h * w4_ref[...], axis=1, keepdims=True)
    e = jnp.exp(s - jnp.max(s))
    z = jnp.sum(e)
    ap = jnp.sum(e * h, axis=0, keepdims=True) / z

    mean = jnp.sum(h, axis=0, keepdims=True) * (1.0 / _T)
    last = h[511:512, :]

    M16 = h.astype(jnp.bfloat16).reshape(_TB, _LM)
    W = wall_ref

    mx = jnp.max(h, axis=0, keepdims=True)

    Y0 = jnp.dot(M16[:, 0:1024], W[:, 0:128], preferred_element_type=f32)
    Y1 = jnp.dot(M16[:, 1024:2048], W[:, 128:256], preferred_element_type=f32)
    Y2 = jnp.dot(M16[:, 2048:3072], W[:, 256:384], preferred_element_type=f32)
    Y3 = jnp.dot(M16[:, 3072:4096], W[:, 384:512], preferred_element_type=f32)

    Ym = Y0[:, 0:64] + Y1[:, 0:64] + Y2[:, 0:64] + Y3[:, 0:64]
    Yn = Y0[:, 64:128] + Y1[:, 64:128]
    Yp = Y2[:, 64:128] + Y3[:, 64:128]
    zrow = jnp.zeros((1, _HID), f32)
    c4 = (Ym
          + jnp.concatenate([zrow, Yp[0:127]], axis=0)
          + jnp.concatenate([Yn[1:128], zrow], axis=0))
    g = _gelu(c4 + cb_ref[...])
    cv = jnp.mean(g, axis=0, keepdims=True)

    pool_ref[0] = jnp.concatenate([mean, last, mx, ap, cv], axis=1)


def _finish_body(pool_ref,
                 rawp_ref, bigw_ref, bvec_ref, hw_ref, hb_ref,
                 a0w1_ref, a0b1_ref, a0w2_ref, a0b2_ref,
                 a1w1_ref, a1b1_ref, a1w2_ref, a1b2_ref,
                 a2w1_ref, a2b1_ref, a2w2_ref, a2b2_ref,
                 a3w1_ref, a3b1_ref, a3w2_ref, a3b2_ref,
                 a4wt_ref, a4b_ref, out_ref):
    f32 = jnp.float32
    cf = jnp.dot(rawp_ref[...], bigw_ref[...], preferred_element_type=f32) + bvec_ref[...]
    logits = jnp.dot(_gelu(cf), hw_ref[...], preferred_element_type=f32) + hb_ref[...]
    ee = jnp.exp(logits - jnp.max(logits, axis=1, keepdims=True))
    wts = ee / jnp.sum(ee, axis=1, keepdims=True)

    def mlp(x, w1t_ref, b1_ref, w2t_ref, b2_ref):
        hmid = _gelu(jnp.dot(x, w1t_ref[...], preferred_element_type=f32) + b1_ref[...])
        return jnp.dot(hmid, w2t_ref[...], preferred_element_type=f32) + b2_ref[...]

    pool = pool_ref[:, 0, :]
    o0 = mlp(pool[:, 0:1024], a0w1_ref, a0b1_ref, a0w2_ref, a0b2_ref)
    o1 = mlp(pool[:, 1024:2048], a1w1_ref, a1b1_ref, a1w2_ref, a1b2_ref)
    o2 = mlp(pool[:, 2048:3072], a2w1_ref, a2b1_ref, a2w2_ref, a2b2_ref)
    o3 = mlp(pool[:, 3072:4096], a3w1_ref, a3b1_ref, a3w2_ref, a3b2_ref)
    o4 = jnp.dot(pool[:, 4096:4160], a4wt_ref[...], preferred_element_type=f32) + a4b_ref[...]

    out_ref[...] = (wts[:, 0:1] * o0 + wts[:, 1:2] * o1 + wts[:, 2:3] * o2
                    + wts[:, 3:4] * o3 + wts[:, 4:5] * o4)


def kernel(hidden_states, raw_input, router_conv_w, router_conv_b, router_head_w, router_head_b,
           a0_w1, a0_b1, a0_w2, a0_b2, a1_w1, a1_b1, a1_w2, a1_b2, a2_w1, a2_b1, a2_w2, a2_b2,
           a3_attn_w, a3_attn_b, a3_w1, a3_b1, a3_w2, a3_b2, a4_conv_w, a4_conv_b, a4_out_w, a4_out_b):
    f32 = jnp.float32

    w4 = a3_attn_w
    wt = jnp.transpose(a4_conv_w, (1, 2, 0))
    wall = jnp.concatenate(
        [wt[:, 2], wt[:, 6], wt[:, 3], wt[:, 7],
         wt[:, 4], wt[:, 0], wt[:, 5], wt[:, 1]], axis=1).astype(jnp.bfloat16)
    cb = a4_conv_b[None, :]

    rawp = jnp.pad(raw_input, ((0, 0), (8, 8)))
    w2d = router_conv_w[:, 0, :]
    e0 = np.eye(33, 32, dtype=np.float32)
    e1 = np.eye(33, 32, k=-1, dtype=np.float32)
    bigw = jnp.kron(e0, w2d[:, :16].T) + jnp.kron(e1, w2d[:, 16:].T)
    bvec = jnp.tile(router_conv_b, 32)[None, :]
    sel = np.zeros((512, 64), dtype=np.float32)
    ii = np.arange(512)
    sel[ii, (ii % 16) * 4 + (ii // 16) // 8] = 1.0 / 8.0
    hw = jnp.dot(jnp.asarray(sel), router_head_w.T)
    hb = router_head_b[None, :]

    pool = pl.pallas_call(
        _stream_body,
        grid=(_B,),
        in_specs=[
            pl.BlockSpec((1, _T, _D), lambda b: (b, 0, 0)),
            pl.BlockSpec((1, _D), lambda b: (0, 0)),
            pl.BlockSpec((_D, 8 * _HID), lambda b: (0, 0)),
            pl.BlockSpec((1, _HID), lambda b: (0, 0)),
        ],
        out_specs=pl.BlockSpec((1, 1, 4 * _D + _HID), lambda b: (b, 0, 0)),
        out_shape=jax.ShapeDtypeStruct((_B, 1, 4 * _D + _HID), f32),
    )(hidden_states, w4, wall, cb)

    out = pl.pallas_call(
        _finish_body,
        out_shape=jax.ShapeDtypeStruct((_B, _OUT), f32),
    )(pool, rawp, bigw, bvec, hw, hb,
      a0_w1.T, a0_b1[None, :], a0_w2.T, a0_b2[None, :],
      a1_w1.T, a1_b1[None, :], a1_w2.T, a1_b2[None, :],
      a2_w1.T, a2_b1[None, :], a2_w2.T, a2_b2[None, :],
      a3_w1.T, a3_b1[None, :], a3_w2.T, a3_b2[None, :],
      a4_out_w.T, a4_out_b[None, :])

    return out


# speedup vs baseline: 1.1024x; 1.1024x over previous
import jax
import jax.numpy as jnp
import numpy as np
from jax.experimental import pallas as pl

_B = 128
_T = 512
_D = 1024
_OUT = 96
_HID = 64
_K = 5
_INLEN = 512
_TB = 128
_LM = 4096
_BPB = 2


def _gelu(x):
    return 0.5 * x * (1.0 + jax.lax.erf(x * 0.7071067811865476))


def _stream_body(m_ref, w4_ref, wall_ref, cb_ref, pool_ref):
    for i in range(_BPB):
        _stream_one(m_ref, w4_ref, wall_ref, cb_ref, pool_ref, i)


def _stream_one(m_ref, w4_ref, wall_ref, cb_ref, pool_ref, i):
    h = m_ref[i]
    f32 = jnp.float32

    s = jnp.sum(h * w4_ref[...], axis=1, keepdims=True)
    e = jnp.exp(s - jnp.max(s))
    z = jnp.sum(e)
    ap = jnp.sum(e * h, axis=0, keepdims=True) / z

    mean = jnp.sum(h, axis=0, keepdims=True) * (1.0 / _T)
    last = h[511:512, :]

    M16 = h.astype(jnp.bfloat16).reshape(_TB, _LM)
    W = wall_ref

    mx = jnp.max(h, axis=0, keepdims=True)

    Y0 = jnp.dot(M16[:, 0:1024], W[:, 0:128], preferred_element_type=f32)
    Y1 = jnp.dot(M16[:, 1024:2048], W[:, 128:256], preferred_element_type=f32)
    Y2 = jnp.dot(M16[:, 2048:3072], W[:, 256:384], preferred_element_type=f32)
    Y3 = jnp.dot(M16[:, 3072:4096], W[:, 384:512], preferred_element_type=f32)

    Ym = Y0[:, 0:64] + Y1[:, 0:64] + Y2[:, 0:64] + Y3[:, 0:64]
    Yn = Y0[:, 64:128] + Y1[:, 64:128]
    Yp = Y2[:, 64:128] + Y3[:, 64:128]
    zrow = jnp.zeros((1, _HID), f32)
    c4 = (Ym
          + jnp.concatenate([zrow, Yp[0:127]], axis=0)
          + jnp.concatenate([Yn[1:128], zrow], axis=0))
    g = _gelu(c4 + cb_ref[...])
    cv = jnp.mean(g, axis=0, keepdims=True)

    pool_ref[i] = jnp.concatenate([mean, last, mx, ap, cv], axis=1)


def _finish_body(pool_ref,
                 rawp_ref, bigw_ref, bvec_ref, hw_ref, hb_ref,
                 a0w1_ref, a0b1_ref, a0w2_ref, a0b2_ref,
                 a1w1_ref, a1b1_ref, a1w2_ref, a1b2_ref,
                 a2w1_ref, a2b1_ref, a2w2_ref, a2b2_ref,
                 a3w1_ref, a3b1_ref, a3w2_ref, a3b2_ref,
                 a4wt_ref, a4b_ref, out_ref):
    f32 = jnp.float32
    cf = jnp.dot(rawp_ref[...], bigw_ref[...], preferred_element_type=f32) + bvec_ref[...]
    logits = jnp.dot(_gelu(cf), hw_ref[...], preferred_element_type=f32) + hb_ref[...]
    ee = jnp.exp(logits - jnp.max(logits, axis=1, keepdims=True))
    wts = ee / jnp.sum(ee, axis=1, keepdims=True)

    def mlp(x, w1t_ref, b1_ref, w2t_ref, b2_ref):
        hmid = _gelu(jnp.dot(x, w1t_ref[...], preferred_element_type=f32) + b1_ref[...])
        return jnp.dot(hmid, w2t_ref[...], preferred_element_type=f32) + b2_ref[...]

    pool = pool_ref[:, 0, :]
    o0 = mlp(pool[:, 0:1024], a0w1_ref, a0b1_ref, a0w2_ref, a0b2_ref)
    o1 = mlp(pool[:, 1024:2048], a1w1_ref, a1b1_ref, a1w2_ref, a1b2_ref)
    o2 = mlp(pool[:, 2048:3072], a2w1_ref, a2b1_ref, a2w2_ref, a2b2_ref)
    o3 = mlp(pool[:, 3072:4096], a3w1_ref, a3b1_ref, a3w2_ref, a3b2_ref)
    o4 = jnp.dot(pool[:, 4096:4160], a4wt_ref[...], preferred_element_type=f32) + a4b_ref[...]

    out_ref[...] = (wts[:, 0:1] * o0 + wts[:, 1:2] * o1 + wts[:, 2:3] * o2
                    + wts[:, 3:4] * o3 + wts[:, 4:5] * o4)


def kernel(hidden_states, raw_input, router_conv_w, router_conv_b, router_head_w, router_head_b,
           a0_w1, a0_b1, a0_w2, a0_b2, a1_w1, a1_b1, a1_w2, a1_b2, a2_w1, a2_b1, a2_w2, a2_b2,
           a3_attn_w, a3_attn_b, a3_w1, a3_b1, a3_w2, a3_b2, a4_conv_w, a4_conv_b, a4_out_w, a4_out_b):
    f32 = jnp.float32

    w4 = a3_attn_w
    wt = jnp.transpose(a4_conv_w, (1, 2, 0))
    wall = jnp.concatenate(
        [wt[:, 2], wt[:, 6], wt[:, 3], wt[:, 7],
         wt[:, 4], wt[:, 0], wt[:, 5], wt[:, 1]], axis=1).astype(jnp.bfloat16)
    cb = a4_conv_b[None, :]

    rawp = jnp.pad(raw_input, ((0, 0), (8, 8)))
    w2d = router_conv_w[:, 0, :]
    e0 = np.eye(33, 32, dtype=np.float32)
    e1 = np.eye(33, 32, k=-1, dtype=np.float32)
    bigw = jnp.kron(e0, w2d[:, :16].T) + jnp.kron(e1, w2d[:, 16:].T)
    bvec = jnp.tile(router_conv_b, 32)[None, :]
    sel = np.zeros((512, 64), dtype=np.float32)
    ii = np.arange(512)
    sel[ii, (ii % 16) * 4 + (ii // 16) // 8] = 1.0 / 8.0
    hw = jnp.dot(jnp.asarray(sel), router_head_w.T)
    hb = router_head_b[None, :]

    pool = pl.pallas_call(
        _stream_body,
        grid=(_B // _BPB,),
        in_specs=[
            pl.BlockSpec((_BPB, _T, _D), lambda b: (b, 0, 0)),
            pl.BlockSpec((1, _D), lambda b: (0, 0)),
            pl.BlockSpec((_D, 8 * _HID), lambda b: (0, 0)),
            pl.BlockSpec((1, _HID), lambda b: (0, 0)),
        ],
        out_specs=pl.BlockSpec((_BPB, 1, 4 * _D + _HID), lambda b: (b, 0, 0)),
        out_shape=jax.ShapeDtypeStruct((_B, 1, 4 * _D + _HID), f32),
    )(hidden_states, w4, wall, cb)

    out = pl.pallas_call(
        _finish_body,
        out_shape=jax.ShapeDtypeStruct((_B, _OUT), f32),
    )(pool, rawp, bigw, bvec, hw, hb,
      a0_w1.T, a0_b1[None, :], a0_w2.T, a0_b2[None, :],
      a1_w1.T, a1_b1[None, :], a1_w2.T, a1_b2[None, :],
      a2_w1.T, a2_b1[None, :], a2_w2.T, a2_b2[None, :],
      a3_w1.T, a3_b1[None, :], a3_w2.T, a3_b2[None, :],
      a4_out_w.T, a4_out_b[None, :])

    return out


# speedup vs baseline: 1.1177x; 1.0139x over previous
import jax
import jax.numpy as jnp
import numpy as np
from jax.experimental import pallas as pl

_B = 128
_T = 512
_D = 1024
_OUT = 96
_HID = 64
_K = 5
_INLEN = 512
_TB = 128
_LM = 4096
_BPB = 4


def _gelu(x):
    return 0.5 * x * (1.0 + jax.lax.erf(x * 0.7071067811865476))


def _stream_body(m_ref, w4_ref, wall_ref, cb_ref, pool_ref):
    for i in range(_BPB):
        _stream_one(m_ref, w4_ref, wall_ref, cb_ref, pool_ref, i)


def _stream_one(m_ref, w4_ref, wall_ref, cb_ref, pool_ref, i):
    h = m_ref[i]
    f32 = jnp.float32

    s = jnp.sum(h * w4_ref[...], axis=1, keepdims=True)
    e = jnp.exp(s - jnp.max(s))
    z = jnp.sum(e)
    ap = jnp.sum(e * h, axis=0, keepdims=True) / z

    mean = jnp.sum(h, axis=0, keepdims=True) * (1.0 / _T)
    last = h[511:512, :]

    M16 = h.astype(jnp.bfloat16).reshape(_TB, _LM)
    W = wall_ref

    mx = jnp.max(h, axis=0, keepdims=True)

    Y0 = jnp.dot(M16[:, 0:1024], W[:, 0:128], preferred_element_type=f32)
    Y1 = jnp.dot(M16[:, 1024:2048], W[:, 128:256], preferred_element_type=f32)
    Y2 = jnp.dot(M16[:, 2048:3072], W[:, 256:384], preferred_element_type=f32)
    Y3 = jnp.dot(M16[:, 3072:4096], W[:, 384:512], preferred_element_type=f32)

    Ym = Y0[:, 0:64] + Y1[:, 0:64] + Y2[:, 0:64] + Y3[:, 0:64]
    Yn = Y0[:, 64:128] + Y1[:, 64:128]
    Yp = Y2[:, 64:128] + Y3[:, 64:128]
    zrow = jnp.zeros((1, _HID), f32)
    c4 = (Ym
          + jnp.concatenate([zrow, Yp[0:127]], axis=0)
          + jnp.concatenate([Yn[1:128], zrow], axis=0))
    g = _gelu(c4 + cb_ref[...])
    cv = jnp.mean(g, axis=0, keepdims=True)

    pool_ref[i] = jnp.concatenate([mean, last, mx, ap, cv], axis=1)


def _finish_body(pool_ref,
                 rawp_ref, bigw_ref, bvec_ref, hw_ref, hb_ref,
                 a0w1_ref, a0b1_ref, a0w2_ref, a0b2_ref,
                 a1w1_ref, a1b1_ref, a1w2_ref, a1b2_ref,
                 a2w1_ref, a2b1_ref, a2w2_ref, a2b2_ref,
                 a3w1_ref, a3b1_ref, a3w2_ref, a3b2_ref,
                 a4wt_ref, a4b_ref, out_ref):
    f32 = jnp.float32
    cf = jnp.dot(rawp_ref[...], bigw_ref[...], preferred_element_type=f32) + bvec_ref[...]
    logits = jnp.dot(_gelu(cf), hw_ref[...], preferred_element_type=f32) + hb_ref[...]
    ee = jnp.exp(logits - jnp.max(logits, axis=1, keepdims=True))
    wts = ee / jnp.sum(ee, axis=1, keepdims=True)

    def mlp(x, w1t_ref, b1_ref, w2t_ref, b2_ref):
        hmid = _gelu(jnp.dot(x, w1t_ref[...], preferred_element_type=f32) + b1_ref[...])
        return jnp.dot(hmid, w2t_ref[...], preferred_element_type=f32) + b2_ref[...]

    pool = pool_ref[:, 0, :]
    o0 = mlp(pool[:, 0:1024], a0w1_ref, a0b1_ref, a0w2_ref, a0b2_ref)
    o1 = mlp(pool[:, 1024:2048], a1w1_ref, a1b1_ref, a1w2_ref, a1b2_ref)
    o2 = mlp(pool[:, 2048:3072], a2w1_ref, a2b1_ref, a2w2_ref, a2b2_ref)
    o3 = mlp(pool[:, 3072:4096], a3w1_ref, a3b1_ref, a3w2_ref, a3b2_ref)
    o4 = jnp.dot(pool[:, 4096:4160], a4wt_ref[...], preferred_element_type=f32) + a4b_ref[...]

    out_ref[...] = (wts[:, 0:1] * o0 + wts[:, 1:2] * o1 + wts[:, 2:3] * o2
                    + wts[:, 3:4] * o3 + wts[:, 4:5] * o4)


def kernel(hidden_states, raw_input, router_conv_w, router_conv_b, router_head_w, router_head_b,
           a0_w1, a0_b1, a0_w2, a0_b2, a1_w1, a1_b1, a1_w2, a1_b2, a2_w1, a2_b1, a2_w2, a2_b2,
           a3_attn_w, a3_attn_b, a3_w1, a3_b1, a3_w2, a3_b2, a4_conv_w, a4_conv_b, a4_out_w, a4_out_b):
    f32 = jnp.float32

    w4 = a3_attn_w
    wt = jnp.transpose(a4_conv_w, (1, 2, 0))
    wall = jnp.concatenate(
        [wt[:, 2], wt[:, 6], wt[:, 3], wt[:, 7],
         wt[:, 4], wt[:, 0], wt[:, 5], wt[:, 1]], axis=1).astype(jnp.bfloat16)
    cb = a4_conv_b[None, :]

    rawp = jnp.pad(raw_input, ((0, 0), (8, 8)))
    w2d = router_conv_w[:, 0, :]
    e0 = np.eye(33, 32, dtype=np.float32)
    e1 = np.eye(33, 32, k=-1, dtype=np.float32)
    bigw = jnp.kron(e0, w2d[:, :16].T) + jnp.kron(e1, w2d[:, 16:].T)
    bvec = jnp.tile(router_conv_b, 32)[None, :]
    sel = np.zeros((512, 64), dtype=np.float32)
    ii = np.arange(512)
    sel[ii, (ii % 16) * 4 + (ii // 16) // 8] = 1.0 / 8.0
    hw = jnp.dot(jnp.asarray(sel), router_head_w.T)
    hb = router_head_b[None, :]

    pool = pl.pallas_call(
        _stream_body,
        grid=(_B // _BPB,),
        in_specs=[
            pl.BlockSpec((_BPB, _T, _D), lambda b: (b, 0, 0)),
            pl.BlockSpec((1, _D), lambda b: (0, 0)),
            pl.BlockSpec((_D, 8 * _HID), lambda b: (0, 0)),
            pl.BlockSpec((1, _HID), lambda b: (0, 0)),
        ],
        out_specs=pl.BlockSpec((_BPB, 1, 4 * _D + _HID), lambda b: (b, 0, 0)),
        out_shape=jax.ShapeDtypeStruct((_B, 1, 4 * _D + _HID), f32),
    )(hidden_states, w4, wall, cb)

    out = pl.pallas_call(
        _finish_body,
        out_shape=jax.ShapeDtypeStruct((_B, _OUT), f32),
    )(pool, rawp, bigw, bvec, hw, hb,
      a0_w1.T, a0_b1[None, :], a0_w2.T, a0_b2[None, :],
      a1_w1.T, a1_b1[None, :], a1_w2.T, a1_b2[None, :],
      a2_w1.T, a2_b1[None, :], a2_w2.T, a2_b2[None, :],
      a3_w1.T, a3_b1[None, :], a3_w2.T, a3_b2[None, :],
      a4_out_w.T, a4_out_b[None, :])

    return out
